# baseline (speedup 1.0000x reference)
import jax
import jax.numpy as jnp
from jax.experimental import pallas as pl
from jax.experimental.pallas import tpu as pltpu
from jax.experimental.pallas import tpu_sc as plsc

_GW = 128


def _gather_pair_rows(tw2, cw2, tgt_idx, ctx_idx, neg_idx):
    D2 = tw2.shape[1]
    Bt = tgt_idx.shape[1]
    Bc = ctx_idx.shape[1]
    Bn = neg_idx.shape[1]
    mesh = plsc.VectorSubcoreMesh(core_axis_name="core", subcore_axis_name="subcore")

    @pl.kernel(
        out_type=(
            jax.ShapeDtypeStruct((Bt, D2), tw2.dtype),
            jax.ShapeDtypeStruct((Bc, D2), cw2.dtype),
            jax.ShapeDtypeStruct((Bn, D2), cw2.dtype),
        ),
        mesh=mesh,
        compiler_params=pltpu.CompilerParams(use_tc_tiling_on_sc=False),
    )
    def k(twt_hbm, cwt_hbm, ti_hbm, ci_hbm, ni_hbm, t_out, c_out, n_out):
        def run(table_hbm, idx_hbm, out_hbm, n_rows):
            def body(i_vmem, o_vmem):
                pltpu.sync_copy(table_hbm.at[i_vmem.at[0]], o_vmem)

            pltpu.emit_pipeline(
                body,
                grid=(n_rows // _GW,),
                in_specs=[pl.BlockSpec((1, _GW), index_map=lambda i: (0, i))],
                out_specs=[pl.BlockSpec((_GW, D2), index_map=lambda i: (i, 0))],
                core_axis_name=("core", "subcore"),
                dimension_semantics=(pltpu.PARALLEL,),
            )(idx_hbm, out_hbm)

        run(twt_hbm, ti_hbm, t_out, Bt)
        run(cwt_hbm, ci_hbm, c_out, Bc)
        run(cwt_hbm, ni_hbm, n_out, Bn)

    return k(tw2, cw2, tgt_idx, ctx_idx, neg_idx)


def _loss_from_pair_rows(t_emb, c_emb, n_emb, par_t, par_c, par_n):
    B, D2 = t_emb.shape
    D = D2 // 2
    K = n_emb.shape[1]
    BB = 512

    def body(t_ref, c_ref, n_ref, pt_ref, pc_ref, pn_ref, o_ref):
        i = pl.program_id(0)
        pt = pt_ref[...]
        pc = pc_ref[...]
        pn = pn_ref[...]
        tf = t_ref[...]
        cf = c_ref[...]
        nf = n_ref[...]
        t = tf[:, :D] * (1.0 - pt) + tf[:, D:] * pt
        c = cf[:, :D] * (1.0 - pc) + cf[:, D:] * pc
        n = nf[..., :D] * (1.0 - pn[..., None]) + nf[..., D:] * pn[..., None]
        pos = jnp.sum(t * c, axis=1)
        neg = jnp.sum(n * t[:, None, :], axis=2)
        part = (-jnp.sum(jax.nn.log_sigmoid(pos))
                - jnp.sum(jax.nn.log_sigmoid(-neg)))

        @pl.when(i == 0)
        def _():
            o_ref[...] = jnp.zeros_like(o_ref)

        o_ref[...] += part[None, None]

    res = pl.pallas_call(
        body,
        grid=(B // BB,),
        in_specs=[
            pl.BlockSpec((BB, D2), lambda i: (i, 0)),
            pl.BlockSpec((BB, D2), lambda i: (i, 0)),
            pl.BlockSpec((BB, K, D2), lambda i: (i, 0, 0)),
            pl.BlockSpec((BB, 1), lambda i: (i, 0)),
            pl.BlockSpec((BB, 1), lambda i: (i, 0)),
            pl.BlockSpec((BB, K), lambda i: (i, 0)),
        ],
        out_specs=pl.BlockSpec((1, 1), lambda i: (0, 0)),
        out_shape=jax.ShapeDtypeStruct((1, 1), jnp.float32),
    )(t_emb, c_emb, n_emb, par_t, par_c, par_n)
    return res[0, 0]


def kernel(target, context, negative_samples, target_weight, context_weight):
    B = target.shape[0]
    K = negative_samples.shape[1]
    V, D = target_weight.shape
    tw2 = target_weight.reshape(V // 2, 2 * D)
    cw2 = context_weight.reshape(V // 2, 2 * D)
    t_i = target.astype(jnp.int32)
    c_i = context.astype(jnp.int32)
    n_i = negative_samples.astype(jnp.int32).reshape(-1)
    t_emb, c_emb, n_emb = _gather_pair_rows(
        tw2, cw2,
        (t_i >> 1).reshape(1, B),
        (c_i >> 1).reshape(1, B),
        (n_i >> 1).reshape(1, B * K),
    )
    n_emb = n_emb.reshape(B, K, 2 * D)
    par_t = (t_i & 1).astype(jnp.float32).reshape(B, 1)
    par_c = (c_i & 1).astype(jnp.float32).reshape(B, 1)
    par_n = (n_i & 1).astype(jnp.float32).reshape(B, K)
    return _loss_from_pair_rows(t_emb, c_emb, n_emb, par_t, par_c, par_n) / B
